# baseline (speedup 1.0000x reference)
import functools

import jax
import jax.numpy as jnp
from jax import lax
from jax.experimental import pallas as pl
from jax.experimental.pallas import tpu as pltpu
from jax.experimental.pallas import tpu_sc as plsc

LANES = 16
NUM_CORES = 2
NUM_SUBCORES = 16
NW = NUM_CORES * NUM_SUBCORES
R = 128
NBUF = 4
GA = NBUF - 2


def _encode(xf, pos, embedding, total, S, D):
    b_per_w = total // NW
    n_chunks = b_per_w // R
    n_groups = n_chunks // NBUF
    nvr = D // LANES

    mesh = plsc.VectorSubcoreMesh(core_axis_name="c", subcore_axis_name="s")

    @functools.partial(
        pl.kernel,
        out_type=jax.ShapeDtypeStruct((total, D), jnp.float32),
        mesh=mesh,
        scratch_types=[
            pltpu.VMEM((n_chunks, R), jnp.int32),
            pltpu.VMEM((S, D), jnp.float32),
            pltpu.VMEM((NBUF, R, D), jnp.float32),
        ]
        + [pltpu.SemaphoreType.DMA] * (2 * NBUF),
    )
    def enc(x_hbm, pos_hbm, emb_hbm, out_hbm, idx_v, pos2_v, rows_v, *sems):
        gsem = sems[:NBUF]
        ssem = sems[NBUF:]
        wid = lax.axis_index("s") * NUM_CORES + lax.axis_index("c")
        row0 = wid * b_per_w

        pltpu.sync_copy(x_hbm.at[pl.ds(wid * n_chunks, n_chunks)], idx_v)
        pltpu.sync_copy(pos_hbm, pos2_v)

        for k in range(GA):
            pltpu.async_copy(emb_hbm.at[idx_v.at[k]], rows_v.at[k], gsem[k])

        def group(g, carry):
            for u in range(NBUF):
                c = g * NBUF + u
                rb = rows_v.at[u]
                pltpu.make_async_copy(
                    emb_hbm.at[idx_v.at[c]], rb, gsem[u]
                ).wait()

                p = lax.rem(c * R, S)

                def addrow(r, carry2, rb=rb, p=p):
                    pr = p + r
                    pr = lax.select(pr >= S, pr - S, pr)
                    pvals = [
                        pos2_v[pr, pl.ds(j * LANES, LANES)]
                        for j in range(nvr)
                    ]
                    for j in range(nvr):
                        plsc.addupdate(
                            rb.at[r, pl.ds(j * LANES, LANES)], pvals[j]
                        )
                    return carry2

                lax.fori_loop(0, R, addrow, 0, unroll=2)

                b2 = (u + GA) % NBUF
                ob2 = out_hbm.at[pl.ds(row0 + (c - 2) * R, R)]

                @pl.when(c >= 2)
                def _(b2=b2, ob2=ob2):
                    pltpu.make_async_copy(rows_v.at[b2], ob2, ssem[b2]).wait()

                pltpu.async_copy(
                    rb, out_hbm.at[pl.ds(row0 + c * R, R)], ssem[u]
                )

                @pl.when(c + GA < n_chunks)
                def _(c=c, b2=b2):
                    pltpu.async_copy(
                        emb_hbm.at[idx_v.at[c + GA]], rows_v.at[b2], gsem[b2]
                    )

            return carry

        lax.fori_loop(0, n_groups, group, 0)

        for c in (n_chunks - 2, n_chunks - 1):
            b = c % NBUF
            pltpu.make_async_copy(
                rows_v.at[b], out_hbm.at[pl.ds(row0 + c * R, R)], ssem[b]
            ).wait()

    return enc(xf, pos, embedding)


def kernel(x, embedding, positional_encoding):
    B, S = x.shape
    V, D = embedding.shape
    total = B * S
    xf = x.reshape(total // R, R).astype(jnp.int32)
    pos = positional_encoding[:S]
    out = _encode(xf, pos, embedding, total, S, D)
    return out.reshape(B, S, D)


# speedup vs baseline: 1.0629x; 1.0629x over previous
import functools

import jax
import jax.numpy as jnp
from jax import lax
from jax.experimental import pallas as pl
from jax.experimental.pallas import tpu as pltpu
from jax.experimental.pallas import tpu_sc as plsc

LANES = 16
NUM_CORES = 2
NUM_SUBCORES = 16
NW = NUM_CORES * NUM_SUBCORES
R = 80
NBUF = 5
GA = NBUF - 2


def _encode(xf, pos, embedding, total, S, D):
    b_per_w = total // NW
    n_chunks = b_per_w // R
    n_groups = n_chunks // NBUF
    PB = S + R - (S % R if S % R else R)
    nvr = D // LANES

    mesh = plsc.VectorSubcoreMesh(core_axis_name="c", subcore_axis_name="s")

    @functools.partial(
        pl.kernel,
        out_type=jax.ShapeDtypeStruct((total, D), jnp.float32),
        mesh=mesh,
        scratch_types=[
            pltpu.VMEM((n_chunks, R), jnp.int32),
            pltpu.VMEM((PB, D), jnp.float32),
            pltpu.VMEM((NBUF, R, D), jnp.float32),
        ]
        + [pltpu.SemaphoreType.DMA] * (2 * NBUF),
    )
    def enc(x_hbm, pos_hbm, emb_hbm, out_hbm, idx_v, pos2_v, rows_v, *sems):
        gsem = sems[:NBUF]
        ssem = sems[NBUF:]
        wid = lax.axis_index("s") * NUM_CORES + lax.axis_index("c")
        row0 = wid * b_per_w

        pltpu.sync_copy(x_hbm.at[pl.ds(wid * n_chunks, n_chunks)], idx_v)
        pltpu.sync_copy(pos_hbm.at[pl.ds(0, S)], pos2_v.at[pl.ds(0, S)])
        pltpu.sync_copy(pos_hbm.at[pl.ds(0, PB - S)], pos2_v.at[pl.ds(S, PB - S)])

        H = R // 2
        for k in range(GA):
            for h in range(2):
                pltpu.async_copy(
                    emb_hbm.at[idx_v.at[k, pl.ds(h * H, H)]],
                    rows_v.at[k, pl.ds(h * H, H)],
                    gsem[k],
                )

        def group(g, carry):
            for u in range(NBUF):
                c = g * NBUF + u
                rb = rows_v.at[u]
                pltpu.make_async_copy(
                    emb_hbm.at[idx_v.at[c]], rb, gsem[u]
                ).wait()

                p = lax.rem(c * R, S)

                def addrow(r, carry2, rb=rb, p=p):
                    pvals = [
                        pos2_v[p + r, pl.ds(j * LANES, LANES)]
                        for j in range(nvr)
                    ]
                    for j in range(nvr):
                        plsc.addupdate(
                            rb.at[r, pl.ds(j * LANES, LANES)], pvals[j]
                        )
                    return carry2

                lax.fori_loop(0, R, addrow, 0, unroll=2)

                b2 = (u + GA) % NBUF
                ob2 = out_hbm.at[pl.ds(row0 + (c - 2) * R, R)]

                @pl.when(c >= 2)
                def _(b2=b2, ob2=ob2):
                    pltpu.make_async_copy(rows_v.at[b2], ob2, ssem[b2]).wait()

                pltpu.async_copy(
                    rb, out_hbm.at[pl.ds(row0 + c * R, R)], ssem[u]
                )

                @pl.when(c + GA < n_chunks)
                def _(c=c, b2=b2):
                    for h in range(2):
                        pltpu.async_copy(
                            emb_hbm.at[idx_v.at[c + GA, pl.ds(h * H, H)]],
                            rows_v.at[b2, pl.ds(h * H, H)],
                            gsem[b2],
                        )

            return carry

        lax.fori_loop(0, n_groups, group, 0)

        for c in (n_chunks - 2, n_chunks - 1):
            b = c % NBUF
            pltpu.make_async_copy(
                rows_v.at[b], out_hbm.at[pl.ds(row0 + c * R, R)], ssem[b]
            ).wait()

    return enc(xf, pos, embedding)


def kernel(x, embedding, positional_encoding):
    B, S = x.shape
    V, D = embedding.shape
    total = B * S
    xf = x.reshape(total // R, R).astype(jnp.int32)
    out = _encode(xf, positional_encoding, embedding, total, S, D)
    return out.reshape(B, S, D)


# speedup vs baseline: 1.0646x; 1.0016x over previous
import functools

import jax
import jax.numpy as jnp
from jax import lax
from jax.experimental import pallas as pl
from jax.experimental.pallas import tpu as pltpu
from jax.experimental.pallas import tpu_sc as plsc

LANES = 16
NUM_CORES = 2
NUM_SUBCORES = 16
NW = NUM_CORES * NUM_SUBCORES
R = 80
NBUF = 5
GA = NBUF - 2


def _encode(xf, pos_full, embedding, total, S, D):
    b_per_w = total // NW
    n_chunks = b_per_w // R
    n_groups = n_chunks // NBUF
    PB = S + R - (S % R if S % R else R)
    nvr = D // LANES

    mesh = plsc.VectorSubcoreMesh(core_axis_name="c", subcore_axis_name="s")

    @functools.partial(
        pl.kernel,
        out_type=jax.ShapeDtypeStruct((total, D), jnp.float32),
        mesh=mesh,
        scratch_types=[
            pltpu.VMEM((n_chunks, R), jnp.int32),
            pltpu.VMEM((PB, D), jnp.float32),
            pltpu.VMEM((NBUF, R, D), jnp.float32),
        ]
        + [pltpu.SemaphoreType.DMA] * (2 * NBUF),
    )
    def enc(x_hbm, pos_hbm, emb_hbm, out_hbm, idx_v, pos2_v, rows_v, *sems):
        gsem = sems[:NBUF]
        ssem = sems[NBUF:]
        wid = lax.axis_index("s") * NUM_CORES + lax.axis_index("c")
        row0 = wid * b_per_w

        pltpu.sync_copy(x_hbm.at[pl.ds(wid * n_chunks, n_chunks)], idx_v)
        pltpu.sync_copy(pos_hbm.at[pl.ds(0, S)], pos2_v.at[pl.ds(0, S)])
        pltpu.sync_copy(pos_hbm.at[pl.ds(0, PB - S)], pos2_v.at[pl.ds(S, PB - S)])

        H = R // 2
        for k in range(GA):
            for h in range(2):
                pltpu.async_copy(
                    emb_hbm.at[idx_v.at[k, pl.ds(h * H, H)]],
                    rows_v.at[k, pl.ds(h * H, H)],
                    gsem[k],
                )

        def group(g, carry):
            for u in range(NBUF):
                c = g * NBUF + u
                rb = rows_v.at[u]
                pltpu.make_async_copy(
                    emb_hbm.at[idx_v.at[c]], rb, gsem[u]
                ).wait()

                b2 = (u + GA) % NBUF
                ob2 = out_hbm.at[pl.ds(row0 + (c - 2) * R, R)]

                @pl.when(c >= 2)
                def _(b2=b2, ob2=ob2):
                    pltpu.make_async_copy(rows_v.at[b2], ob2, ssem[b2]).wait()

                @pl.when(c + GA < n_chunks)
                def _(c=c, b2=b2):
                    for h in range(2):
                        pltpu.async_copy(
                            emb_hbm.at[idx_v.at[c + GA, pl.ds(h * H, H)]],
                            rows_v.at[b2, pl.ds(h * H, H)],
                            gsem[b2],
                        )

                p = lax.rem(c * R, S)

                def addrow(r, carry2, rb=rb, p=p):
                    pvals = [
                        pos2_v[p + r, pl.ds(j * LANES, LANES)]
                        for j in range(nvr)
                    ]
                    for j in range(nvr):
                        plsc.addupdate(
                            rb.at[r, pl.ds(j * LANES, LANES)], pvals[j]
                        )
                    return carry2

                lax.fori_loop(0, R, addrow, 0, unroll=2)

                pltpu.async_copy(
                    rb, out_hbm.at[pl.ds(row0 + c * R, R)], ssem[u]
                )

            return carry

        lax.fori_loop(0, n_groups, group, 0)

        for c in (n_chunks - 2, n_chunks - 1):
            b = c % NBUF
            pltpu.make_async_copy(
                rows_v.at[b], out_hbm.at[pl.ds(row0 + c * R, R)], ssem[b]
            ).wait()

    return enc(xf, pos_full, embedding)


def kernel(x, embedding, positional_encoding):
    B, S = x.shape
    V, D = embedding.shape
    total = B * S
    xf = x.reshape(total // R, R).astype(jnp.int32)
    out = _encode(xf, positional_encoding, embedding, total, S, D)
    return out.reshape(B, S, D)
